# Initial kernel scaffold; baseline (speedup 1.0000x reference)
import jax
import jax.numpy as jnp
from jax.experimental import pallas as pl


def kernel(x, neigh_x, kernel_self, kernel_neigh, bias):
    raise NotImplementedError("write your pallas kernel here")


# speedup vs baseline: 20.8431x; 20.8431x over previous
import functools

import jax
import jax.numpy as jnp
from jax import lax
from jax.experimental import pallas as pl
from jax.experimental.pallas import tpu as pltpu
from jax.experimental.pallas import tpu_sc as plsc

_N = 10000
_DEG = 32
_D = 128
_UNITS = 128
_LANES = 16
_NC = 2
_NS = 16
_NW = _NC * _NS
_CHUNK = 8
_NCHUNKS = _N // _CHUNK


def _oems_pairs(n):
    pairs = []

    def _sort(lo, m):
        if m > 1:
            k = m // 2
            _sort(lo, k)
            _sort(lo + k, k)
            _merge(lo, m, 1)

    def _merge(lo, m, r):
        step = r * 2
        if step < m:
            _merge(lo, m, step)
            _merge(lo + r, m, step)
            for i in range(lo + r, lo + m - r, step):
                pairs.append((i, i + r))
        else:
            pairs.append((lo, lo + r))

    _sort(0, n)
    return pairs


_PAIRS16 = _oems_pairs(16)


def _median32(vals):
    a = list(vals[:16])
    b = list(vals[16:])
    for i, j in _PAIRS16:
        lo = jnp.minimum(a[i], a[j])
        a[j] = jnp.maximum(a[i], a[j])
        a[i] = lo
        lo = jnp.minimum(b[i], b[j])
        b[j] = jnp.maximum(b[i], b[j])
        b[i] = lo
    cand = [jnp.maximum(a[i], b[15 - i]) for i in range(16)]
    while len(cand) > 1:
        cand = [jnp.minimum(cand[2 * i], cand[2 * i + 1])
                for i in range(len(cand) // 2)]
    return cand[0]


def _sc_median_body(neigh_hbm, med_hbm, buf, obuf):
    wid = lax.axis_index("s") * _NC + lax.axis_index("c")
    nk = (_NCHUNKS - wid + (_NW - 1)) // _NW

    def chunk_body(k, carry):
        base = (wid + k * _NW) * _CHUNK
        pltpu.sync_copy(neigh_hbm.at[pl.ds(base, _CHUNK)], buf)

        def node_body(nd, c2):
            def fg_body(fg, c3):
                col = fg * _LANES
                vals = [buf[nd, s, pl.ds(col, _LANES)] for s in range(_DEG)]
                obuf[nd, pl.ds(col, _LANES)] = _median32(vals)
                return c3

            return lax.fori_loop(0, _D // _LANES, fg_body, c2)

        lax.fori_loop(0, _CHUNK, node_body, 0)
        pltpu.sync_copy(obuf, med_hbm.at[pl.ds(base, _CHUNK)])
        return carry

    lax.fori_loop(0, nk, chunk_body, 0)


_sc_median = functools.partial(
    pl.kernel,
    out_type=jax.ShapeDtypeStruct((_N, _D), jnp.float32),
    mesh=plsc.VectorSubcoreMesh(core_axis_name="c", subcore_axis_name="s",
                                num_cores=_NC, num_subcores=_NS),
    scratch_types=[
        pltpu.VMEM((_CHUNK, _DEG, _D), jnp.float32),
        pltpu.VMEM((_CHUNK, _D), jnp.float32),
    ],
)(_sc_median_body)


_BN = 400


def _tc_body(x_ref, med_ref, ws_ref, wn_ref, b_ref, o_ref):
    acc = jnp.dot(x_ref[...], ws_ref[...], preferred_element_type=jnp.float32)
    acc = acc + jnp.dot(med_ref[...], wn_ref[...],
                        preferred_element_type=jnp.float32)
    o_ref[...] = jnp.maximum(acc + b_ref[...], 0.0)


def _tc_matmul(x, med, ws, wn, bias2):
    return pl.pallas_call(
        _tc_body,
        grid=(_N // _BN,),
        in_specs=[
            pl.BlockSpec((_BN, _D), lambda i: (i, 0)),
            pl.BlockSpec((_BN, _D), lambda i: (i, 0)),
            pl.BlockSpec((_D, _UNITS), lambda i: (0, 0)),
            pl.BlockSpec((_D, _UNITS), lambda i: (0, 0)),
            pl.BlockSpec((1, _UNITS), lambda i: (0, 0)),
        ],
        out_specs=pl.BlockSpec((_BN, _UNITS), lambda i: (i, 0)),
        out_shape=jax.ShapeDtypeStruct((_N, _UNITS), jnp.float32),
    )(x, med, ws, wn, bias2)


def kernel(x, neigh_x, kernel_self, kernel_neigh, bias):
    med = _sc_median(neigh_x)
    return _tc_matmul(x, med, kernel_self, kernel_neigh,
                      bias.reshape(1, _UNITS))


# speedup vs baseline: 27.1999x; 1.3050x over previous
import functools

import jax
import jax.numpy as jnp
from jax import lax
from jax.experimental import pallas as pl
from jax.experimental.pallas import tpu as pltpu
from jax.experimental.pallas import tpu_sc as plsc

_N = 10000
_DEG = 32
_D = 128
_UNITS = 128
_LANES = 16
_NC = 2
_NS = 16
_NW = _NC * _NS
_CHUNK = 8
_NCHUNKS = _N // _CHUNK


def _oems_pairs(n):
    pairs = []

    def _sort(lo, m):
        if m > 1:
            k = m // 2
            _sort(lo, k)
            _sort(lo + k, k)
            _merge(lo, m, 1)

    def _merge(lo, m, r):
        step = r * 2
        if step < m:
            _merge(lo, m, step)
            _merge(lo + r, m, step)
            for i in range(lo + r, lo + m - r, step):
                pairs.append((i, i + r))
        else:
            pairs.append((lo, lo + r))

    _sort(0, n)
    return pairs


_PAIRS16 = _oems_pairs(16)


def _median32(vals):
    a = list(vals[:16])
    b = list(vals[16:])
    for i, j in _PAIRS16:
        lo = jnp.minimum(a[i], a[j])
        a[j] = jnp.maximum(a[i], a[j])
        a[i] = lo
        lo = jnp.minimum(b[i], b[j])
        b[j] = jnp.maximum(b[i], b[j])
        b[i] = lo
    cand = [jnp.maximum(a[i], b[15 - i]) for i in range(16)]
    while len(cand) > 1:
        cand = [jnp.minimum(cand[2 * i], cand[2 * i + 1])
                for i in range(len(cand) // 2)]
    return cand[0]


def _sc_median_body(neigh_hbm, med_hbm, buf0, buf1, obuf, sem0, sem1):
    wid = lax.axis_index("s") * _NC + lax.axis_index("c")
    nk = (_NCHUNKS - wid + (_NW - 1)) // _NW
    bufs = (buf0, buf1)
    sems = (sem0, sem1)

    def _in_copy(k, b):
        base = (wid + k * _NW) * _CHUNK
        return pltpu.make_async_copy(
            neigh_hbm.at[pl.ds(base, _CHUNK)], bufs[b], sems[b])

    _in_copy(0, 0).start()
    _in_copy(1, 1).start()

    def _compute(k, b):
        buf = bufs[b]

        def node_body(nd, c):
            for fg in range(_D // _LANES):
                col = fg * _LANES
                vals = [buf[nd, s, pl.ds(col, _LANES)] for s in range(_DEG)]
                obuf[nd, pl.ds(col, _LANES)] = _median32(vals)
            return c

        lax.fori_loop(0, _CHUNK, node_body, 0)
        base = (wid + k * _NW) * _CHUNK
        pltpu.sync_copy(obuf, med_hbm.at[pl.ds(base, _CHUNK)])

    def pair_body(p, c):
        for b in range(2):
            k = 2 * p + b

            def slot(k=k, b=b):
                _in_copy(k, b).wait()
                _compute(k, b)

                def start_next():
                    _in_copy(k + 2, b).start()

                pl.when(k + 2 < nk)(start_next)

            pl.when(k < nk)(slot)
        return c

    lax.fori_loop(0, 20, pair_body, 0)


_sc_median = functools.partial(
    pl.kernel,
    out_type=jax.ShapeDtypeStruct((_N, _D), jnp.float32),
    mesh=plsc.VectorSubcoreMesh(core_axis_name="c", subcore_axis_name="s",
                                num_cores=_NC, num_subcores=_NS),
    scratch_types=[
        pltpu.VMEM((_CHUNK, _DEG, _D), jnp.float32),
        pltpu.VMEM((_CHUNK, _DEG, _D), jnp.float32),
        pltpu.VMEM((_CHUNK, _D), jnp.float32),
        pltpu.SemaphoreType.DMA,
        pltpu.SemaphoreType.DMA,
    ],
)(_sc_median_body)


_BN = 400


def _tc_body(x_ref, med_ref, ws_ref, wn_ref, b_ref, o_ref):
    acc = jnp.dot(x_ref[...], ws_ref[...], preferred_element_type=jnp.float32)
    acc = acc + jnp.dot(med_ref[...], wn_ref[...],
                        preferred_element_type=jnp.float32)
    o_ref[...] = jnp.maximum(acc + b_ref[...], 0.0)


def _tc_matmul(x, med, ws, wn, bias2):
    return pl.pallas_call(
        _tc_body,
        grid=(_N // _BN,),
        in_specs=[
            pl.BlockSpec((_BN, _D), lambda i: (i, 0)),
            pl.BlockSpec((_BN, _D), lambda i: (i, 0)),
            pl.BlockSpec((_D, _UNITS), lambda i: (0, 0)),
            pl.BlockSpec((_D, _UNITS), lambda i: (0, 0)),
            pl.BlockSpec((1, _UNITS), lambda i: (0, 0)),
        ],
        out_specs=pl.BlockSpec((_BN, _UNITS), lambda i: (i, 0)),
        out_shape=jax.ShapeDtypeStruct((_N, _UNITS), jnp.float32),
    )(x, med, ws, wn, bias2)


def kernel(x, neigh_x, kernel_self, kernel_neigh, bias):
    med = _sc_median(neigh_x)
    return _tc_matmul(x, med, kernel_self, kernel_neigh,
                      bias.reshape(1, _UNITS))
